# baseline (speedup 1.0000x reference)
import functools

import numpy as np
import jax
import jax.numpy as jnp
from jax import lax
from jax.experimental import pallas as pl
from jax.experimental.pallas import tpu as pltpu
from jax.experimental.pallas import tpu_sc as plsc

LANES = 16


def _bf16_bits_np(a: np.ndarray) -> np.ndarray:
    bits = a.astype(np.float32).view(np.uint32)
    return (bits + 0x7FFF + ((bits >> 16) & 1)) >> 16


def _pack2_np(lo: np.ndarray, hi: np.ndarray) -> np.ndarray:
    return (((_bf16_bits_np(hi) << 16) | _bf16_bits_np(lo))
            .astype(np.uint32).view(np.int32))


def _pack_pairs_np(a: np.ndarray) -> np.ndarray:
    R, D = a.shape
    blk = a.reshape(R, D // 32, 2, 16)
    return _pack2_np(blk[:, :, 0, :], blk[:, :, 1, :]).reshape(R, D // 2)


def _pack_pairs_jnp(a):
    R, D = a.shape
    blk = a.reshape(R, D // 32, 2, 16)
    bits = lax.bitcast_convert_type(blk.astype(jnp.bfloat16), jnp.uint16)
    bits = bits.astype(jnp.uint32)
    w = (bits[:, :, 1, :] << 16) | bits[:, :, 0, :]
    return lax.bitcast_convert_type(w, jnp.int32).reshape(R, D // 2)


def _sc_info():
    try:
        info = plsc.get_sparse_core_info()
        return info.num_cores, info.num_subcores
    except Exception:
        return 2, 16


_CHUNK = 64
_NSLOT = 2
_HIMASK = np.int32(-65536)


@functools.lru_cache(maxsize=4)
def _pos_tables(L: int, D: int, NW: int, C: int):
    jmax = (D - 1) // 2
    b_per_w = L // NW
    n_chunks = b_per_w // C
    w_full = np.zeros((D // 2,), np.float64)
    j = np.arange(jmax, dtype=np.float64)
    w_full[:jmax] = 1.0 / np.power(10000.0, 2.0 * j / D)

    row0 = (np.arange(NW)[:, None] * b_per_w
            + np.arange(n_chunks)[None, :] * C)
    ang = row0[..., None] * w_full[None, None, :]
    pm_seed = np.zeros((NW, n_chunks, D), np.float32)
    pm_seed[..., 0 : 2 * jmax : 2] = np.sin(ang[..., :jmax])
    pm_seed[..., 1 : 2 * jmax : 2] = np.cos(ang[..., :jmax])
    swapped = pm_seed.reshape(NW, n_chunks, D // 2, 2)[..., ::-1].reshape(
        NW, n_chunks, D)
    seed_pk = _pack2_np(pm_seed, swapped)

    r = np.arange(C, dtype=np.float64)[:, None]
    angr = r * w_full[None, :]
    rc = np.repeat(np.cos(angr), 2, axis=1).astype(np.float32)
    rs = np.repeat(np.sin(angr), 2, axis=1).astype(np.float32)
    sign = np.where(np.arange(D) % 2 == 0, 1.0, -1.0).astype(np.float32)
    rot_pk = _pack2_np(rc, rs * sign[None, :])
    return jnp.asarray(seed_pk), jnp.asarray(rot_pk)


def _widen(w):
    lo = lax.bitcast_convert_type(lax.shift_left(w, 16), jnp.float32)
    hi = lax.bitcast_convert_type(lax.bitwise_and(w, _HIMASK), jnp.float32)
    return lo, hi


@functools.lru_cache(maxsize=4)
def _build_kernel(L: int, V: int, D: int):
    NC, NS = _sc_info()
    NW = NC * NS
    b_per_w = L // NW
    C = _CHUNK
    n_chunks = b_per_w // C
    Dw = D // 2
    assert L % NW == 0 and b_per_w % C == 0 and D % 32 == 0

    mesh = plsc.VectorSubcoreMesh(core_axis_name="c", subcore_axis_name="s")

    @functools.partial(
        pl.kernel,
        mesh=mesh,
        out_type=jax.ShapeDtypeStruct((L, D), jnp.float32),
        scratch_types=[
            pltpu.VMEM((n_chunks, C), jnp.int32),
            pltpu.VMEM((V * Dw,), jnp.int32),
            pltpu.VMEM((n_chunks, D), jnp.int32),
            pltpu.VMEM((C, D), jnp.int32),
            pltpu.VMEM((_NSLOT, C, D), jnp.float32),
            pltpu.SemaphoreType.DMA((_NSLOT,)),
        ],
    )
    def emb(x_hbm, table_hbm, seed_hbm, rot_hbm, out_hbm,
            idx_v, table_v, seed_v, rot_v, ob_v, osem):
        wid = lax.axis_index("s") * NC + lax.axis_index("c")
        base = wid * b_per_w

        pltpu.sync_copy(x_hbm.at[wid], idx_v)
        pltpu.sync_copy(seed_hbm.at[wid], seed_v)
        pltpu.sync_copy(rot_hbm, rot_v)
        pltpu.sync_copy(table_hbm, table_v)

        out_cp = {}
        for ci in range(n_chunks):
            slot = ci % _NSLOT
            if ci - _NSLOT >= 0:
                out_cp[ci - _NSLOT].wait()

            def do_group(g, _):
                gbase = pl.multiple_of(g * LANES, LANES)
                idx_vec = idx_v[ci, pl.ds(gbase, LANES)]
                for r16 in range(LANES):
                    addr0 = idx_vec[r16] * Dw
                    row = gbase + r16

                    @plsc.parallel_loop(0, D // 32, unroll=4)
                    def _(k):
                        wcol = pl.multiple_of(k * LANES, LANES)
                        t0, t1 = _widen(table_v[pl.ds(addr0 + wcol, LANES)])
                        col = pl.multiple_of(k * 32, 32)
                        s0, w0 = _widen(seed_v[ci, pl.ds(col, LANES)])
                        rc0, rs0 = _widen(rot_v[row, pl.ds(col, LANES)])
                        s1, w1 = _widen(seed_v[ci, pl.ds(col + LANES, LANES)])
                        rc1, rs1 = _widen(rot_v[row, pl.ds(col + LANES, LANES)])
                        ob_v[slot, row, pl.ds(col, LANES)] = (
                            t0 + s0 * rc0 + w0 * rs0)
                        ob_v[slot, row, pl.ds(col + LANES, LANES)] = (
                            t1 + s1 * rc1 + w1 * rs1)
                return 0

            lax.fori_loop(0, C // LANES, do_group, 0)
            row0 = base + ci * C
            out_cp[ci] = pltpu.async_copy(ob_v.at[slot],
                                          out_hbm.at[pl.ds(row0, C)],
                                          osem.at[slot])
        for ci in range(max(0, n_chunks - _NSLOT), n_chunks):
            out_cp[ci].wait()

    return emb


def kernel(x, wordlist):
    L = x.shape[0]
    V, D = wordlist.shape
    NC, NS = _sc_info()
    NW = NC * NS
    seed_pk, rot_pk = _pos_tables(L, D, NW, _CHUNK)
    emb = _build_kernel(L, V, D)
    x_grp = x.astype(jnp.int32).reshape(NW, L // NW // _CHUNK, _CHUNK)
    wt = _pack_pairs_jnp(wordlist).reshape(-1)
    return emb(x_grp, wt, seed_pk, rot_pk)


# speedup vs baseline: 1.1418x; 1.1418x over previous
import functools

import numpy as np
import jax
import jax.numpy as jnp
from jax import lax
from jax.experimental import pallas as pl
from jax.experimental.pallas import tpu as pltpu
from jax.experimental.pallas import tpu_sc as plsc

LANES = 16


def _bf16_bits_np(a: np.ndarray) -> np.ndarray:
    bits = a.astype(np.float32).view(np.uint32)
    return (bits + 0x7FFF + ((bits >> 16) & 1)) >> 16


def _pack2_np(lo: np.ndarray, hi: np.ndarray) -> np.ndarray:
    return (((_bf16_bits_np(hi) << 16) | _bf16_bits_np(lo))
            .astype(np.uint32).view(np.int32))


def _pack_pairs_np(a: np.ndarray) -> np.ndarray:
    R, D = a.shape
    blk = a.reshape(R, D // 32, 2, 16)
    return _pack2_np(blk[:, :, 0, :], blk[:, :, 1, :]).reshape(R, D // 2)


def _pack_pairs_jnp(a):
    R, D = a.shape
    blk = a.reshape(R, D // 32, 2, 16)
    bits = lax.bitcast_convert_type(blk.astype(jnp.bfloat16), jnp.uint16)
    bits = bits.astype(jnp.uint32)
    w = (bits[:, :, 1, :] << 16) | bits[:, :, 0, :]
    return lax.bitcast_convert_type(w, jnp.int32).reshape(R, D // 2)


def _sc_info():
    try:
        info = plsc.get_sparse_core_info()
        return info.num_cores, info.num_subcores
    except Exception:
        return 2, 16


_CHUNK = 64
_NSLOT = 2
_HIMASK = np.int32(-65536)


@functools.lru_cache(maxsize=4)
def _pos_tables(L: int, D: int, NW: int, C: int):
    jmax = (D - 1) // 2
    b_per_w = L // NW
    n_chunks = b_per_w // C
    w_full = np.zeros((D // 2,), np.float64)
    j = np.arange(jmax, dtype=np.float64)
    w_full[:jmax] = 1.0 / np.power(10000.0, 2.0 * j / D)

    row0 = (np.arange(NW)[:, None] * b_per_w
            + np.arange(n_chunks)[None, :] * C)
    ang = row0[..., None] * w_full[None, None, :]
    pm_seed = np.zeros((NW, n_chunks, D), np.float32)
    pm_seed[..., 0 : 2 * jmax : 2] = np.sin(ang[..., :jmax])
    pm_seed[..., 1 : 2 * jmax : 2] = np.cos(ang[..., :jmax])
    swapped = pm_seed.reshape(NW, n_chunks, D // 2, 2)[..., ::-1].reshape(
        NW, n_chunks, D)
    seed_pk = _pack2_np(pm_seed, swapped)

    r = np.arange(C, dtype=np.float64)[:, None]
    angr = r * w_full[None, :]
    rc = np.repeat(np.cos(angr), 2, axis=1).astype(np.float32)
    rs = np.repeat(np.sin(angr), 2, axis=1).astype(np.float32)
    sign = np.where(np.arange(D) % 2 == 0, 1.0, -1.0).astype(np.float32)
    rot_pk = _pack2_np(rc, rs * sign[None, :])
    return jnp.asarray(seed_pk), jnp.asarray(rot_pk)


def _widen(w):
    lo = lax.bitcast_convert_type(lax.shift_left(w, 16), jnp.float32)
    hi = lax.bitcast_convert_type(lax.bitwise_and(w, _HIMASK), jnp.float32)
    return lo, hi


@functools.lru_cache(maxsize=4)
def _build_kernel(L: int, V: int, D: int):
    NC, NS = _sc_info()
    NW = NC * NS
    b_per_w = L // NW
    C = _CHUNK
    n_chunks = b_per_w // C
    Dw = D // 2
    assert L % NW == 0 and b_per_w % C == 0 and D % 32 == 0

    mesh = plsc.VectorSubcoreMesh(core_axis_name="c", subcore_axis_name="s")

    @functools.partial(
        pl.kernel,
        mesh=mesh,
        out_type=jax.ShapeDtypeStruct((L, D), jnp.float32),
        scratch_types=[
            pltpu.VMEM((n_chunks, C), jnp.int32),
            pltpu.VMEM((V * Dw,), jnp.int32),
            pltpu.VMEM((n_chunks, D), jnp.int32),
            pltpu.VMEM((C, D), jnp.int32),
            pltpu.VMEM((_NSLOT, C, D), jnp.float32),
            pltpu.SMEM((C,), jnp.int32),
            pltpu.SemaphoreType.DMA((_NSLOT,)),
        ],
    )
    def emb(x_hbm, table_hbm, seed_hbm, rot_hbm, out_hbm,
            idx_v, table_v, seed_v, rot_v, ob_v, addr_s, osem):
        wid = lax.axis_index("s") * NC + lax.axis_index("c")
        base = wid * b_per_w

        pltpu.sync_copy(x_hbm.at[wid], idx_v)
        pltpu.sync_copy(seed_hbm.at[wid], seed_v)
        pltpu.sync_copy(rot_hbm, rot_v)
        pltpu.sync_copy(table_hbm, table_v)

        out_cp = {}
        for ci in range(n_chunks):
            slot = ci % _NSLOT
            if ci - _NSLOT >= 0:
                out_cp[ci - _NSLOT].wait()

            def fill_addr(g, _):
                gbase = pl.multiple_of(g * LANES, LANES)
                idx_vec = idx_v[ci, pl.ds(gbase, LANES)]
                for r16 in range(LANES):
                    addr_s[gbase + r16] = idx_vec[r16] * Dw
                return 0

            lax.fori_loop(0, C // LANES, fill_addr, 0)

            @plsc.parallel_loop(0, C * (D // 32), unroll=4)
            def _(q):
                row = lax.shift_right_logical(q, 4)
                k = lax.bitwise_and(q, 15)
                addr0 = addr_s[row]
                wcol = pl.multiple_of(k * LANES, LANES)
                t0, t1 = _widen(table_v[pl.ds(addr0 + wcol, LANES)])
                col = pl.multiple_of(k * 32, 32)
                s0, w0 = _widen(seed_v[ci, pl.ds(col, LANES)])
                rc0, rs0 = _widen(rot_v[row, pl.ds(col, LANES)])
                s1, w1 = _widen(seed_v[ci, pl.ds(col + LANES, LANES)])
                rc1, rs1 = _widen(rot_v[row, pl.ds(col + LANES, LANES)])
                ob_v[slot, row, pl.ds(col, LANES)] = t0 + s0 * rc0 + w0 * rs0
                ob_v[slot, row, pl.ds(col + LANES, LANES)] = (
                    t1 + s1 * rc1 + w1 * rs1)
            row0 = base + ci * C
            out_cp[ci] = pltpu.async_copy(ob_v.at[slot],
                                          out_hbm.at[pl.ds(row0, C)],
                                          osem.at[slot])
        for ci in range(max(0, n_chunks - _NSLOT), n_chunks):
            out_cp[ci].wait()

    return emb


def kernel(x, wordlist):
    L = x.shape[0]
    V, D = wordlist.shape
    NC, NS = _sc_info()
    NW = NC * NS
    seed_pk, rot_pk = _pos_tables(L, D, NW, _CHUNK)
    emb = _build_kernel(L, V, D)
    x_grp = x.astype(jnp.int32).reshape(NW, L // NW // _CHUNK, _CHUNK)
    wt = _pack_pairs_jnp(wordlist).reshape(-1)
    return emb(x_grp, wt, seed_pk, rot_pk)


# speedup vs baseline: 1.2026x; 1.0532x over previous
import functools

import numpy as np
import jax
import jax.numpy as jnp
from jax import lax
from jax.experimental import pallas as pl
from jax.experimental.pallas import tpu as pltpu
from jax.experimental.pallas import tpu_sc as plsc

LANES = 16


def _bf16_bits_np(a: np.ndarray) -> np.ndarray:
    bits = a.astype(np.float32).view(np.uint32)
    return (bits + 0x7FFF + ((bits >> 16) & 1)) >> 16


def _pack_pairs_np(a: np.ndarray) -> np.ndarray:
    R, D = a.shape
    blk = a.reshape(R, D // 32, 2, 16)
    lo = _bf16_bits_np(blk[:, :, 0, :])
    hi = _bf16_bits_np(blk[:, :, 1, :])
    return ((hi << 16) | lo).astype(np.uint32).view(np.int32).reshape(R, D // 2)


def _pack_pairs_jnp(a):
    R, D = a.shape
    blk = a.reshape(R, D // 32, 2, 16)
    bits = lax.bitcast_convert_type(blk.astype(jnp.bfloat16), jnp.uint16)
    bits = bits.astype(jnp.uint32)
    w = (bits[:, :, 1, :] << 16) | bits[:, :, 0, :]
    return lax.bitcast_convert_type(w, jnp.int32).reshape(R, D // 2)


@functools.lru_cache(maxsize=4)
def _pos_matrix_packed(L: int, D: int):
    jmax = (D - 1) // 2
    i = np.arange(L, dtype=np.float32)[:, None]
    j = np.arange(jmax, dtype=np.float32)[None, :]
    angle = (i / np.power(10000.0, 2.0 * j / np.float32(D))).astype(np.float32)
    pm = np.zeros((L, D), dtype=np.float32)
    pm[:, 0 : 2 * jmax : 2] = np.sin(angle)
    pm[:, 1 : 2 * jmax : 2] = np.cos(angle)
    return jnp.asarray(_pack_pairs_np(pm))


def _sc_info():
    try:
        info = plsc.get_sparse_core_info()
        return info.num_cores, info.num_subcores
    except Exception:
        return 2, 16


_CHUNK = 64
_NSLOT = 2
_PRE = 1
_HIMASK = np.int32(-65536)


def _widen(w):
    lo = lax.bitcast_convert_type(lax.shift_left(w, 16), jnp.float32)
    hi = lax.bitcast_convert_type(lax.bitwise_and(w, _HIMASK), jnp.float32)
    return lo, hi


@functools.lru_cache(maxsize=4)
def _build_kernel(L: int, V: int, D: int):
    NC, NS = _sc_info()
    NW = NC * NS
    b_per_w = L // NW
    C = _CHUNK
    n_chunks = b_per_w // C
    Dw = D // 2
    assert L % NW == 0 and b_per_w % C == 0 and D % 32 == 0

    mesh = plsc.VectorSubcoreMesh(core_axis_name="c", subcore_axis_name="s")

    @functools.partial(
        pl.kernel,
        mesh=mesh,
        out_type=jax.ShapeDtypeStruct((L, D), jnp.float32),
        scratch_types=[
            pltpu.VMEM((n_chunks, C), jnp.int32),
            pltpu.VMEM((V * Dw,), jnp.int32),
            pltpu.VMEM((_NSLOT, C, Dw), jnp.int32),
            pltpu.VMEM((_NSLOT, C, D), jnp.float32),
            pltpu.SMEM((C,), jnp.int32),
            pltpu.SemaphoreType.DMA((_NSLOT,)),
            pltpu.SemaphoreType.DMA((_NSLOT,)),
        ],
    )
    def emb(x_hbm, table_hbm, pm_hbm, out_hbm, idx_v, table_v, pm_v, ob_v,
            addr_s, psem, osem):
        wid = lax.axis_index("s") * NC + lax.axis_index("c")
        base = wid * b_per_w

        def fire_pm(ci):
            slot = ci % _NSLOT
            row0 = base + ci * C
            return pltpu.async_copy(pm_hbm.at[pl.ds(row0, C)],
                                    pm_v.at[slot], psem.at[slot])

        pm_cp = {ci: fire_pm(ci) for ci in range(min(_PRE, n_chunks))}
        pltpu.sync_copy(x_hbm.at[wid], idx_v)
        pltpu.sync_copy(table_hbm, table_v)

        out_cp = {}
        for ci in range(n_chunks):
            slot = ci % _NSLOT
            nxt = ci + _PRE
            if nxt < n_chunks:
                if nxt - _NSLOT >= 0:
                    out_cp[nxt - _NSLOT].wait()
                pm_cp[nxt] = fire_pm(nxt)
            pm_cp.pop(ci).wait()

            def fill_addr(g, _):
                gbase = pl.multiple_of(g * LANES, LANES)
                idx_vec = idx_v[ci, pl.ds(gbase, LANES)]
                for r16 in range(LANES):
                    addr_s[gbase + r16] = idx_vec[r16] * Dw
                return 0

            lax.fori_loop(0, C // LANES, fill_addr, 0)

            @plsc.parallel_loop(0, C * (D // 32), unroll=4)
            def _(q):
                row = lax.shift_right_logical(q, 4)
                k = lax.bitwise_and(q, 15)
                addr0 = addr_s[row]
                wcol = pl.multiple_of(k * LANES, LANES)
                t0, t1 = _widen(table_v[pl.ds(addr0 + wcol, LANES)])
                p0, p1 = _widen(pm_v[slot, row, pl.ds(wcol, LANES)])
                col = pl.multiple_of(k * 32, 32)
                ob_v[slot, row, pl.ds(col, LANES)] = t0 + p0
                ob_v[slot, row, pl.ds(col + LANES, LANES)] = t1 + p1
            row0 = base + ci * C
            out_cp[ci] = pltpu.async_copy(ob_v.at[slot],
                                          out_hbm.at[pl.ds(row0, C)],
                                          osem.at[slot])
        for ci in range(max(0, n_chunks - _NSLOT), n_chunks):
            out_cp[ci].wait()

    return emb


def kernel(x, wordlist):
    L = x.shape[0]
    V, D = wordlist.shape
    NC, NS = _sc_info()
    NW = NC * NS
    pm = _pos_matrix_packed(L, D)
    emb = _build_kernel(L, V, D)
    x_grp = x.astype(jnp.int32).reshape(NW, L // NW // _CHUNK, _CHUNK)
    wt = _pack_pairs_jnp(wordlist).reshape(-1)
    return emb(x_grp, wt, pm)
